# speedup vs baseline: 1.5898x; 1.5898x over previous
import jax
import jax.numpy as jnp
from jax.experimental import pallas as pl

_NTOKEN = 1000
_BBLK = 8


def _onehot_body(x_ref, out_ref):
    x = x_ref[...]
    iota = jax.lax.broadcasted_iota(jnp.int32, out_ref.shape, 2)
    out_ref[...] = (x[:, :, None] == iota).astype(jnp.float32)


def kernel(x):
    B, L = x.shape
    grid = (B // _BBLK,)
    return pl.pallas_call(
        _onehot_body,
        grid=grid,
        in_specs=[pl.BlockSpec((_BBLK, L), lambda i: (i, 0))],
        out_specs=pl.BlockSpec((_BBLK, L, _NTOKEN), lambda i: (i, 0, 0)),
        out_shape=jax.ShapeDtypeStruct((B, L, _NTOKEN), jnp.float32),
    )(x)


# speedup vs baseline: 1.7743x; 1.1160x over previous
import jax
import jax.numpy as jnp
from jax.experimental import pallas as pl

_NTOKEN = 1000
_BBLK = 32


def _onehot_body(x_ref, out_ref):
    x = x_ref[...]
    iota = jax.lax.broadcasted_iota(jnp.int32, out_ref.shape, 2)
    out_ref[...] = (x[:, :, None] == iota).astype(jnp.float32)


def kernel(x):
    B, L = x.shape
    grid = (B // _BBLK,)
    return pl.pallas_call(
        _onehot_body,
        grid=grid,
        in_specs=[pl.BlockSpec((_BBLK, L), lambda i: (i, 0))],
        out_specs=pl.BlockSpec((_BBLK, L, _NTOKEN), lambda i: (i, 0, 0)),
        out_shape=jax.ShapeDtypeStruct((B, L, _NTOKEN), jnp.float32),
    )(x)
